# baseline (speedup 1.0000x reference)
import functools

import jax
import jax.numpy as jnp
from jax import lax
from jax.experimental import pallas as pl
from jax.experimental.pallas import tpu as pltpu
from jax.experimental.pallas import tpu_sc as plsc

BATCH = 4096
SEQ = 200
DIM = 64
NUM_WORKERS = 32
B_PER_W = BATCH // NUM_WORKERS
CHUNK = 100
CHUNKS_PER_B = SEQ // CHUNK
N_CHUNKS = B_PER_W * CHUNKS_PER_B
LANES = 16
VECS = DIM // LANES


def _bow_body(idx_hbm, table_hbm, out_hbm, idx_v, rows0, rows1, out_v,
              sem0, sem1):
    wid = lax.axis_index("s") * 2 + lax.axis_index("c")

    pltpu.sync_copy(idx_hbm.at[pl.ds(wid * N_CHUNKS, N_CHUNKS)], idx_v)

    pltpu.async_copy(table_hbm.at[idx_v.at[0]], rows0, sem0)
    pltpu.async_copy(table_hbm.at[idx_v.at[1]], rows1, sem1)

    def accumulate(buf, accs):
        def r_body(r, a):
            return tuple(a[d] + buf[r, pl.ds(d * LANES, LANES)]
                         for d in range(VECS))
        return lax.fori_loop(0, CHUNK, r_body, accs)

    def b_body(b, carry):
        accs = tuple(jnp.zeros((LANES,), jnp.float32) for _ in range(VECS))
        for h, (buf, sem) in enumerate(((rows0, sem0), (rows1, sem1))):
            pltpu.make_async_copy(table_hbm.at[idx_v.at[0]], buf, sem).wait()
            accs = accumulate(buf, accs)
            nxt = 2 * (b + 1) + h

            @pl.when(nxt < N_CHUNKS)
            def _():
                pltpu.async_copy(table_hbm.at[idx_v.at[nxt]], buf, sem)

        for d in range(VECS):
            out_v[b, pl.ds(d * LANES, LANES)] = accs[d]
        return carry

    lax.fori_loop(0, B_PER_W, b_body, 0)

    pltpu.sync_copy(out_v, out_hbm.at[pl.ds(wid * B_PER_W, B_PER_W)])


@functools.partial(
    pl.kernel,
    mesh=plsc.VectorSubcoreMesh(core_axis_name="c", subcore_axis_name="s"),
    out_type=jax.ShapeDtypeStruct((BATCH, DIM), jnp.float32),
    scratch_types=[
        pltpu.VMEM((N_CHUNKS, CHUNK), jnp.int32),
        pltpu.VMEM((CHUNK, DIM), jnp.float32),
        pltpu.VMEM((CHUNK, DIM), jnp.float32),
        pltpu.VMEM((B_PER_W, DIM), jnp.float32),
        pltpu.SemaphoreType.DMA,
        pltpu.SemaphoreType.DMA,
    ],
    compiler_params=pltpu.CompilerParams(use_tc_tiling_on_sc=False),
)
def _bow_sc(idx_hbm, table_hbm, out_hbm, idx_v, rows0, rows1, out_v,
            sem0, sem1):
    _bow_body(idx_hbm, table_hbm, out_hbm, idx_v, rows0, rows1, out_v,
              sem0, sem1)


@jax.jit
def kernel(indices, table):
    idx = indices.astype(jnp.int32).reshape(BATCH * SEQ // CHUNK, CHUNK)
    return _bow_sc(idx, table)


# speedup vs baseline: 1.0699x; 1.0699x over previous
import functools

import jax
import jax.numpy as jnp
from jax import lax
from jax.experimental import pallas as pl
from jax.experimental.pallas import tpu as pltpu
from jax.experimental.pallas import tpu_sc as plsc

BATCH = 4096
SEQ = 200
DIM = 64
NUM_WORKERS = 32
B_PER_W = BATCH // NUM_WORKERS
CHUNK = 100
CHUNKS_PER_B = SEQ // CHUNK
N_CHUNKS = B_PER_W * CHUNKS_PER_B
LANES = 16
VECS = DIM // LANES


def _bow_body(idx_hbm, table_hbm, out_hbm, idx_v, rows0, rows1, out_v,
              sem0, sem1):
    wid = lax.axis_index("s") * 2 + lax.axis_index("c")

    pltpu.sync_copy(idx_hbm.at[pl.ds(wid * B_PER_W, B_PER_W)], idx_v)

    pltpu.async_copy(table_hbm.at[idx_v.at[0]], rows0, sem0)
    pltpu.async_copy(table_hbm.at[idx_v.at[1]], rows1, sem1)

    def accumulate(buf, accs):
        def r_body(r, a):
            a = [x + buf[2 * r, pl.ds(d * LANES, LANES)]
                 for d, x in enumerate(a)]
            return tuple(x + buf[2 * r + 1, pl.ds(d * LANES, LANES)]
                         for d, x in enumerate(a))
        return lax.fori_loop(0, SEQ // 2, r_body, accs)

    def b_body(b, carry):
        buf = (rows0, rows1)
        sems = (sem0, sem1)
        for p, (buf, sem) in enumerate(((rows0, sem0), (rows1, sem1))):
            bb = 2 * b + p
            pltpu.make_async_copy(table_hbm.at[idx_v.at[0]], buf, sem).wait()
            accs = tuple(jnp.zeros((LANES,), jnp.float32)
                         for _ in range(VECS))
            accs = accumulate(buf, accs)
            for d in range(VECS):
                out_v[bb, pl.ds(d * LANES, LANES)] = accs[d]
            nxt = bb + 2

            @pl.when(nxt < B_PER_W)
            def _():
                pltpu.async_copy(table_hbm.at[idx_v.at[nxt]], buf, sem)

        return carry

    lax.fori_loop(0, B_PER_W // 2, b_body, 0)

    pltpu.sync_copy(out_v, out_hbm.at[pl.ds(wid * B_PER_W, B_PER_W)])


@functools.partial(
    pl.kernel,
    mesh=plsc.VectorSubcoreMesh(core_axis_name="c", subcore_axis_name="s"),
    out_type=jax.ShapeDtypeStruct((BATCH, DIM), jnp.float32),
    scratch_types=[
        pltpu.VMEM((B_PER_W, SEQ), jnp.int32),
        pltpu.VMEM((SEQ, DIM), jnp.float32),
        pltpu.VMEM((SEQ, DIM), jnp.float32),
        pltpu.VMEM((B_PER_W, DIM), jnp.float32),
        pltpu.SemaphoreType.DMA,
        pltpu.SemaphoreType.DMA,
    ],
    compiler_params=pltpu.CompilerParams(use_tc_tiling_on_sc=False),
)
def _bow_sc(idx_hbm, table_hbm, out_hbm, idx_v, rows0, rows1, out_v,
            sem0, sem1):
    _bow_body(idx_hbm, table_hbm, out_hbm, idx_v, rows0, rows1, out_v,
              sem0, sem1)


@jax.jit
def kernel(indices, table):
    return _bow_sc(indices.astype(jnp.int32), table)
